# Initial kernel scaffold; baseline (speedup 1.0000x reference)
import jax
import jax.numpy as jnp
from jax.experimental import pallas as pl


def kernel(x_v, x_e, adj_row, adj_col, x_v_batch, emb_table, W_edge_init, b_edge_init, Wih_v2e, Whh_v2e, bih_v2e, bhh_v2e, Wih_e2v, Whh_e2v, bih_e2v, bhh_e2v, W_out, b_out):
    raise NotImplementedError("write your pallas kernel here")


# speedup vs baseline: 3.5090x; 3.5090x over previous
import functools

import jax
import jax.numpy as jnp
from jax import lax
from jax.experimental import pallas as pl
from jax.experimental.pallas import tpu as pltpu
from jax.experimental.pallas import tpu_sc as plsc

D = 128
NC = 2
NS = 16
LANES = 16
BLK = 8192
CHUNK = 128
SUBROWS = BLK // NS


def _make_segsum(n_src_pad, n_dst_pad, nnz_pad, nb):
    nloop = -(-nb // NC)
    mesh = plsc.VectorSubcoreMesh(core_axis_name="c", subcore_axis_name="s")

    @functools.partial(
        pl.kernel,
        out_type=jax.ShapeDtypeStruct((n_dst_pad, D), jnp.float32),
        mesh=mesh,
        scratch_types=[
            pltpu.VMEM((CHUNK,), jnp.int32),
            pltpu.VMEM((CHUNK,), jnp.int32),
            pltpu.VMEM((3 * NS + LANES,), jnp.int32),
            pltpu.VMEM((CHUNK, D), jnp.float32),
            pltpu.VMEM((CHUNK, D), jnp.float32),
            pltpu.VMEM_SHARED((BLK + LANES, D), jnp.float32),
            pltpu.SemaphoreType.DMA,
        ],
    )
    def segsum(src_hbm, gidx_hbm, dloc_hbm, ta_hbm, ts_hbm, te_hbm, zeros_hbm,
               out_hbm, gidx_v, didx_v, bnd_v, rows_v, zero_v, acc,
               sem):
        core = lax.axis_index("c")
        sub = lax.axis_index("s")
        lane = lax.iota(jnp.int32, LANES)

        pltpu.sync_copy(zeros_hbm, zero_v)

        def block_body(i, carry):
            b = i * NC + core
            inblk = b < nb
            plsc.subcore_barrier()

            @pl.when(inblk)
            def _zero():
                for k in range(SUBROWS // CHUNK):
                    pltpu.sync_copy(
                        zero_v, acc.at[pl.ds(sub * SUBROWS + k * CHUNK, CHUNK)])

                @pl.when(sub == 0)
                def _zero_dump():
                    pltpu.sync_copy(zero_v.at[pl.ds(0, LANES)],
                                    acc.at[pl.ds(BLK, LANES)])

            plsc.subcore_barrier()

            @pl.when(inblk)
            def _scatter():
                pltpu.sync_copy(ta_hbm.at[pl.ds(b * NS, NS)],
                                bnd_v.at[pl.ds(0, NS)])
                pltpu.sync_copy(ts_hbm.at[pl.ds(b * NS, NS)],
                                bnd_v.at[pl.ds(NS, NS)])
                pltpu.sync_copy(te_hbm.at[pl.ds(b * NS, NS)],
                                bnd_v.at[pl.ds(2 * NS, NS)])
                a = pl.multiple_of(bnd_v[pl.ds(sub, LANES)][0], 8)
                s = bnd_v[pl.ds(NS + sub, LANES)][0]
                e = bnd_v[pl.ds(2 * NS + sub, LANES)][0]
                nch = (e - a + (CHUNK - 1)) // CHUNK

                def chunk(j, c2):
                    off = pl.multiple_of(a + j * CHUNK, 8)
                    pltpu.sync_copy(gidx_hbm.at[pl.ds(off, CHUNK)], gidx_v)
                    pltpu.sync_copy(dloc_hbm.at[pl.ds(off, CHUNK)], didx_v)
                    for k in range(CHUNK // LANES):
                        pos = off + k * LANES + lane
                        dv = didx_v[pl.ds(k * LANES, LANES)]
                        ok = (pos >= s) & (pos < e)
                        didx_v[pl.ds(k * LANES, LANES)] = jnp.where(
                            ok, dv, jnp.int32(BLK))
                    pltpu.async_copy(src_hbm.at[gidx_v], rows_v, sem).wait()
                    pltpu.sync_copy(rows_v, acc.at[didx_v], add=True)
                    return c2

                lax.fori_loop(0, nch, chunk, 0)

            plsc.subcore_barrier()

            @pl.when(inblk)
            def _copyout():
                for k in range(SUBROWS // CHUNK):
                    r0 = sub * SUBROWS + k * CHUNK
                    pltpu.sync_copy(acc.at[pl.ds(r0, CHUNK)], rows_v)
                    pltpu.sync_copy(rows_v, out_hbm.at[pl.ds(b * BLK + r0,
                                                             CHUNK)])

            return carry

        lax.fori_loop(0, nloop, block_body, 0)

    return segsum


def _make_gather(n_rows_pad, tbl_rows):
    per_w = n_rows_pad // (NC * NS)
    nch = per_w // CHUNK
    mesh = plsc.VectorSubcoreMesh(core_axis_name="c", subcore_axis_name="s")

    @functools.partial(
        pl.kernel,
        out_type=jax.ShapeDtypeStruct((n_rows_pad, D), jnp.float32),
        mesh=mesh,
        scratch_types=[
            pltpu.VMEM((CHUNK,), jnp.int32),
            pltpu.VMEM((CHUNK, D), jnp.float32),
            pltpu.SemaphoreType.DMA,
        ],
    )
    def gather(tbl_hbm, idx_hbm, out_hbm, idx_v, rows_v, sem):
        core = lax.axis_index("c")
        sub = lax.axis_index("s")
        base = (sub * NC + core) * per_w

        def body(j, c2):
            off = base + j * CHUNK
            pltpu.sync_copy(idx_hbm.at[pl.ds(off, CHUNK)], idx_v)
            pltpu.async_copy(tbl_hbm.at[idx_v], rows_v, sem).wait()
            pltpu.sync_copy(rows_v, out_hbm.at[pl.ds(off, CHUNK)])
            return c2

        lax.fori_loop(0, nch, body, 0)

    return gather


RB = 2048


def _lstm_body(has_h, has_c, refs):
    i = 0
    msg_ref = refs[i]; i += 1
    h_ref = None
    c_ref = None
    if has_h:
        h_ref = refs[i]; i += 1
    if has_c:
        c_ref = refs[i]; i += 1
    wih_ref = refs[i]; i += 1
    whh_ref = refs[i]; i += 1
    b_ref = refs[i]; i += 1
    h2_ref, c2_ref = refs[i], refs[i + 1]
    gates = jnp.dot(msg_ref[...], wih_ref[...],
                    preferred_element_type=jnp.float32)
    if has_h:
        gates = gates + jnp.dot(h_ref[...], whh_ref[...],
                                preferred_element_type=jnp.float32)
    gates = gates + b_ref[...]
    gi = jax.nn.sigmoid(gates[:, 0:D])
    gf = jax.nn.sigmoid(gates[:, D:2 * D])
    gg = jnp.tanh(gates[:, 2 * D:3 * D])
    go = jax.nn.sigmoid(gates[:, 3 * D:4 * D])
    if has_c:
        c2 = gf * c_ref[...] + gi * gg
    else:
        c2 = gi * gg
    h2_ref[...] = go * jnp.tanh(c2)
    c2_ref[...] = c2


def _lstm_call(msg, h, c, wih_t, whh_t, bias):
    n = msg.shape[0]
    has_h, has_c = h is not None, c is not None
    row_spec = pl.BlockSpec((RB, D), lambda i: (i, 0))
    in_specs = [row_spec]
    args = [msg]
    if has_h:
        in_specs.append(row_spec)
        args.append(h)
    if has_c:
        in_specs.append(row_spec)
        args.append(c)
    in_specs += [
        pl.BlockSpec((D, 4 * D), lambda i: (0, 0)),
        pl.BlockSpec((D, 4 * D), lambda i: (0, 0)),
        pl.BlockSpec((1, 4 * D), lambda i: (0, 0)),
    ]
    args += [wih_t, whh_t, bias]
    out_shape = [jax.ShapeDtypeStruct((n, D), jnp.float32),
                 jax.ShapeDtypeStruct((n, D), jnp.float32)]
    return pl.pallas_call(
        lambda *refs: _lstm_body(has_h, has_c, refs),
        grid=(n // RB,),
        in_specs=in_specs,
        out_specs=[row_spec, row_spec],
        out_shape=out_shape,
    )(*args)


def _logits_call(h_v_pad, n_v, w_out_t, b_out):
    v = w_out_t.shape[1]
    rb = 2000
    body = lambda h_ref, w_ref, b_ref, o_ref: o_ref.__setitem__(
        ..., jnp.dot(h_ref[...], w_ref[...],
                     preferred_element_type=jnp.float32) + b_ref[...])
    return pl.pallas_call(
        body,
        grid=(n_v // rb,),
        in_specs=[
            pl.BlockSpec((rb, D), lambda i: (i, 0)),
            pl.BlockSpec((D, v), lambda i: (0, 0)),
            pl.BlockSpec((1, v), lambda i: (0, 0)),
        ],
        out_specs=pl.BlockSpec((rb, v), lambda i: (i, 0)),
        out_shape=jax.ShapeDtypeStruct((n_v, v), jnp.float32),
    )(h_v_pad, w_out_t, b_out)


def kernel(x_v, x_e, adj_row, adj_col, x_v_batch, emb_table, W_edge_init,
           b_edge_init, Wih_v2e, Whh_v2e, bih_v2e, bhh_v2e, Wih_e2v, Whh_e2v,
           bih_e2v, bhh_e2v, W_out, b_out):
    n_v = x_v.shape[0]
    n_e = x_e.shape[0]
    nnz = adj_row.shape[0]
    vocab = emb_table.shape[0] - 1
    vpad = -(-n_v // BLK) * BLK
    epad = -(-n_e // BLK) * BLK
    nb_v = vpad // BLK
    nb_e = epad // BLK
    nnz_pad = nnz + 2 * CHUNK

    ar = adj_row.astype(jnp.int32)
    ac = adj_col.astype(jnp.int32)
    dst_e, src_v2e = lax.sort([ar, ac], num_keys=1)
    dst_v, src_e2v = lax.sort([ac, ar], num_keys=1)

    tsub = jnp.arange(NS, dtype=jnp.int32)[None, :]

    def prep(dst_s, src_s, nb):
        dloc = jnp.bitwise_and(dst_s, BLK - 1)
        edges = jnp.arange(nb + 1, dtype=jnp.int32) * BLK
        bstart = jnp.searchsorted(dst_s, edges).astype(jnp.int32)
        s0 = bstart[:-1][:, None]
        cnt = (bstart[1:] - bstart[:-1])[:, None]
        ts = s0 + cnt * tsub // NS
        te = s0 + cnt * (tsub + 1) // NS
        ta = jnp.bitwise_and(ts, ~jnp.int32(7))
        pad = jnp.zeros((nnz_pad - nnz,), jnp.int32)
        return (jnp.concatenate([src_s, pad]), jnp.concatenate([dloc, pad]),
                ta.reshape(-1), ts.reshape(-1), te.reshape(-1))

    g_e, dl_e, ta_e, ts_e, te_e = prep(dst_e, src_v2e, nb_e)
    g_v, dl_v, ta_v, ts_v, te_v = prep(dst_v, src_e2v, nb_v)
    zeros128 = jnp.zeros((CHUNK, D), jnp.float32)

    idx0 = jnp.where(x_v[:, 0] < 0, vocab, x_v[:, 0]).astype(jnp.int32)
    idx0 = jnp.concatenate([idx0, jnp.zeros((vpad - n_v,), jnp.int32)])
    h_v = _make_gather(vpad, emb_table.shape[0])(emb_table, idx0)

    wih_e, whh_e = Wih_v2e.T, Whh_v2e.T
    wih_v, whh_v = Wih_e2v.T, Whh_e2v.T
    b_e = (bih_v2e + bhh_v2e)[None, :]
    b_v = (bih_e2v + bhh_e2v)[None, :]
    edge_h0 = W_edge_init[:, 0] + b_edge_init
    b_e_first = (edge_h0 @ whh_e)[None, :] + b_e

    segsum_e = _make_segsum(vpad, epad, nnz_pad, nb_e)
    segsum_v = _make_segsum(epad, vpad, nnz_pad, nb_v)

    h_e = c_e = c_v = None
    for it in range(3):
        msg_e = segsum_e(h_v, g_e, dl_e, ta_e, ts_e, te_e, zeros128)
        if it == 0:
            h_e, c_e = _lstm_call(msg_e, None, None, wih_e, whh_e, b_e_first)
        else:
            h_e, c_e = _lstm_call(msg_e, h_e, c_e, wih_e, whh_e, b_e)
        msg_v = segsum_v(h_e, g_v, dl_v, ta_v, ts_v, te_v, zeros128)
        if it == 0:
            h_v, c_v = _lstm_call(msg_v, h_v, None, wih_v, whh_v, b_v)
        else:
            h_v, c_v = _lstm_call(msg_v, h_v, c_v, wih_v, whh_v, b_v)

    return _logits_call(h_v, n_v, W_out.T, b_out[None, :])
